# baseline (speedup 1.0000x reference)
import functools

import jax
import jax.numpy as jnp
from jax.experimental import pallas as pl
from jax.experimental.pallas import tpu as pltpu


def _stream_body(x_hbm, w_hbm, b_ref, o_hbm, inb, wv, outb,
                 insem, wsem, outsem, *, C, R):
    base = pl.program_id(0) * C * R

    pltpu.make_async_copy(w_hbm, wv, wsem).start()
    for c in range(C):
        pltpu.make_async_copy(
            x_hbm.at[pl.ds(base + c * R, R), :], inb.at[c], insem.at[c]).start()
    pltpu.make_async_copy(w_hbm, wv, wsem).wait()

    for c in range(C):
        pltpu.make_async_copy(
            x_hbm.at[pl.ds(base + c * R, R), :], inb.at[c], insem.at[c]).wait()
        acc = jax.lax.dot_general(
            inb[c], wv[...],
            dimension_numbers=(((1,), (1,)), ((), ())),
            preferred_element_type=jnp.float32,
        )
        s = c % 2
        if c >= 2:
            pltpu.make_async_copy(
                outb.at[s], o_hbm.at[pl.ds(base + (c - 2) * R, R), :],
                outsem.at[s]).wait()
        outb[s] = acc + b_ref[...]
        pltpu.make_async_copy(
            outb.at[s], o_hbm.at[pl.ds(base + c * R, R), :], outsem.at[s]).start()

    for c in range(max(C - 2, 0), C):
        s = c % 2
        pltpu.make_async_copy(
            outb.at[s], o_hbm.at[pl.ds(base + c * R, R), :], outsem.at[s]).wait()


def _emitter_matmul_body(x_ref, w_ref, b_ref, o_ref):
    acc = jax.lax.dot_general(
        x_ref[...], w_ref[...],
        dimension_numbers=(((1,), (1,)), ((), ())),
        preferred_element_type=jnp.float32,
    )
    o_ref[...] = acc + b_ref[...]


def _emitter_fallback(x, weight, b2, M, K, N):
    tm = min(2048, M)
    blocks = pl.cdiv(M, tm)
    cores = 2 if blocks % 2 == 0 else 1
    inner = blocks // cores
    return pl.pallas_call(
        _emitter_matmul_body,
        out_shape=jax.ShapeDtypeStruct((M, N), x.dtype),
        grid=(cores, inner),
        in_specs=[
            pl.BlockSpec((tm, K), lambda i, j, inner=inner: (i * inner + j, 0)),
            pl.BlockSpec((N, K), lambda i, j: (0, 0)),
            pl.BlockSpec((1, N), lambda i, j: (0, 0)),
        ],
        out_specs=pl.BlockSpec((tm, N), lambda i, j, inner=inner: (i * inner + j, 0)),
        compiler_params=pltpu.CompilerParams(
            dimension_semantics=("parallel", "arbitrary"),
            vmem_limit_bytes=48 * 1024 * 1024,
        ),
    )(x, weight, b2)


def kernel(x, weight, bias):
    M, K = x.shape
    N = weight.shape[0]
    b2 = bias.reshape(1, N)

    R = 1024
    if M % (2 * R) != 0:
        return _emitter_fallback(x, weight, b2, M, K, N)
    C = M // (2 * R)

    return pl.pallas_call(
        functools.partial(_stream_body, C=C, R=R),
        out_shape=jax.ShapeDtypeStruct((M, N), x.dtype),
        grid=(2,),
        in_specs=[
            pl.BlockSpec(memory_space=pl.ANY),
            pl.BlockSpec(memory_space=pl.ANY),
            pl.BlockSpec((1, N), lambda i: (0, 0)),
        ],
        out_specs=pl.BlockSpec(memory_space=pl.ANY),
        scratch_shapes=[
            pltpu.VMEM((C, R, K), jnp.float32),
            pltpu.VMEM((N, K), jnp.float32),
            pltpu.VMEM((2, R, N), jnp.float32),
            pltpu.SemaphoreType.DMA((C,)),
            pltpu.SemaphoreType.DMA,
            pltpu.SemaphoreType.DMA((2,)),
        ],
        compiler_params=pltpu.CompilerParams(
            dimension_semantics=("parallel",),
            vmem_limit_bytes=48 * 1024 * 1024,
        ),
    )(x, weight, b2)


# speedup vs baseline: 1.1173x; 1.1173x over previous
import jax
import jax.numpy as jnp
from jax.experimental import pallas as pl
from jax.experimental.pallas import tpu as pltpu


def _matmul_body(x_ref, w_ref, b_ref, o_ref):
    acc = jax.lax.dot_general(
        x_ref[...], w_ref[...],
        dimension_numbers=(((1,), (1,)), ((), ())),
        preferred_element_type=jnp.float32,
    )
    o_ref[...] = acc + b_ref[...]


def kernel(x, weight, bias):
    M, K = x.shape
    N = weight.shape[0]
    b2 = bias.reshape(1, N)

    tm = min(2048, M)
    grid = (pl.cdiv(M, tm),)
    return pl.pallas_call(
        _matmul_body,
        out_shape=jax.ShapeDtypeStruct((M, N), x.dtype),
        grid=grid,
        in_specs=[
            pl.BlockSpec((tm, K), lambda i: (i, 0)),
            pl.BlockSpec((N, K), lambda i: (0, 0)),
            pl.BlockSpec((1, N), lambda i: (0, 0)),
        ],
        out_specs=pl.BlockSpec((tm, N), lambda i: (i, 0)),
        compiler_params=pltpu.CompilerParams(
            dimension_semantics=("parallel",),
            vmem_limit_bytes=48 * 1024 * 1024,
        ),
    )(x, weight, b2)
